# Initial kernel scaffold; baseline (speedup 1.0000x reference)
import jax
import jax.numpy as jnp
from jax.experimental import pallas as pl


def kernel(embedding, l1_W, l1_b, l2_W, l2_b, node_emb, m1_W, m1_b, m2_W, m2_b, out_W, out_b):
    raise NotImplementedError("write your pallas kernel here")


# speedup vs baseline: 1.9581x; 1.9581x over previous
import functools

import jax
import jax.numpy as jnp
import numpy as np
from jax.experimental import pallas as pl
from jax.experimental.pallas import tpu as pltpu

D = 8
W = 512
NUM_NODES = 1 + D * W
H = 64
EMB = 32
IN_F = 128
B = 256
P = 4

BT = 8


def _build_parent_mats():
    rng = np.random.RandomState(0)
    mats = np.zeros((D - 1, W, W), dtype=np.float32)
    for d in range(2, D + 1):
        start = 2 + (d - 1) * W
        prev = np.arange(start - W, start)
        for w in range(W):
            p = rng.choice(prev, size=P, replace=False)
            mats[d - 2, w, p - (start - W)] = 1.0
    return mats


_G = _build_parent_mats()


def _body(emb, l1_wt, l1_b, l2_wt, l2_b, ne, m1_wt, m1_b, m2_wt, m2_b,
          oww, owr, g, o_main, o_root):
    h = jnp.maximum(jnp.dot(emb[...], l1_wt[...],
                            preferred_element_type=jnp.float32) + l1_b[...], 0.0)
    h = jnp.dot(h, l2_wt[...], preferred_element_type=jnp.float32) + l2_b[...]
    o_root[...] = jnp.sum(h * owr[...], axis=1, keepdims=True)

    pv = jnp.broadcast_to(h[:, None, :], (BT, W, H))
    for d in range(D):
        if d > 0:
            gd = g[d - 1]
            pv = jnp.stack(
                [jnp.dot(gd, pv[b], preferred_element_type=jnp.float32)
                 for b in range(BT)])
        ne_d = jnp.broadcast_to(ne[d][None], (BT, W, EMB))
        x = jnp.concatenate([pv.reshape(BT * W, H),
                             ne_d.reshape(BT * W, EMB)], axis=1)
        t = jnp.maximum(jnp.dot(x, m1_wt[...],
                                preferred_element_type=jnp.float32) + m1_b[...], 0.0)
        v = jnp.dot(t, m2_wt[...], preferred_element_type=jnp.float32) + m2_b[...]
        cur = v.reshape(BT, W, H) + pv
        o_main[:, d, :] = jnp.sum(cur * oww[d][None], axis=2)
        pv = cur


def kernel(embedding, l1_W, l1_b, l2_W, l2_b, node_emb, m1_W, m1_b, m2_W,
           m2_b, out_W, out_b):
    ne_all = node_emb[2:2 + D * W].reshape(D, W, EMB)
    oww = out_W[0, 1:].reshape(D, W, H)
    owr = out_W[0, 0:1, :]

    full = lambda shape: pl.BlockSpec(shape, lambda i: (0,) * len(shape))
    grid = B // BT
    o_main, o_root = pl.pallas_call(
        _body,
        grid=(grid,),
        in_specs=[
            pl.BlockSpec((BT, IN_F), lambda i: (i, 0)),
            full((IN_F, H)), full((1, H)), full((H, H)), full((1, H)),
            full((D, W, EMB)),
            full((H + EMB, H)), full((1, H)), full((H, H)), full((1, H)),
            full((D, W, H)), full((1, H)),
            full((D - 1, W, W)),
        ],
        out_specs=[
            pl.BlockSpec((BT, D, W), lambda i: (i, 0, 0)),
            pl.BlockSpec((BT, 1), lambda i: (i, 0)),
        ],
        out_shape=[
            jax.ShapeDtypeStruct((B, D, W), jnp.float32),
            jax.ShapeDtypeStruct((B, 1), jnp.float32),
        ],
        compiler_params=pltpu.CompilerParams(
            dimension_semantics=("arbitrary",),
        ),
    )(embedding, l1_W.T, l1_b[None], l2_W.T, l2_b[None], ne_all,
      m1_W.T, m1_b[None], m2_W.T, m2_b[None], oww, owr, _G)

    out = jnp.concatenate([o_root, o_main.reshape(B, D * W)], axis=1)
    return out + out_b
